# baseline (speedup 1.0000x reference)
import functools

import jax
import jax.numpy as jnp
from jax import lax
from jax.experimental import pallas as pl
from jax.experimental.pallas import tpu as pltpu
from jax.experimental.pallas import tpu_sc as plsc

N = 16384
NNZ = 262144
BATCH = 64
NC = 2
NS = 16
HB = BATCH // NC
NPT = NNZ // NS
CH = 128
NCH = NPT // CH
NG = 4
GC = NCH // NG
DEPTH = 4
SLAB = N // NS
RC = 128
NRC = SLAB // RC
TB = 512

_mesh = plsc.VectorSubcoreMesh(core_axis_name="c", subcore_axis_name="s")


def _pre_body(z_ref, u_ref, y_ref, zu_ref, yt_ref):
    zu = z_ref[...] * u_ref[...]
    zu_ref[...] = zu.T
    yt_ref[...] = y_ref[...].T


_raw_spec = pl.BlockSpec((HB, TB), lambda c, j: (c, j))
_tsp_spec = pl.BlockSpec((TB, HB), lambda c, j: (c * (N // TB) + j, 0))

_tc_pre = pl.pallas_call(
    _pre_body,
    grid=(NC, N // TB),
    in_specs=[_raw_spec, _raw_spec, _raw_spec],
    out_specs=[_tsp_spec, _tsp_spec],
    out_shape=[
        jax.ShapeDtypeStruct((NC * N, HB), jnp.float32),
        jax.ShapeDtypeStruct((NC * N, HB), jnp.float32),
    ],
)


def _post_body(es_ref, z_ref, u_ref, o_ref):
    o_ref[...] = z_ref[...] - u_ref[...] * es_ref[...].T


_tc_post = pl.pallas_call(
    _post_body,
    grid=(NC, N // TB),
    in_specs=[_tsp_spec, _raw_spec, _raw_spec],
    out_specs=_raw_spec,
    out_shape=jax.ShapeDtypeStruct((BATCH, N), jnp.float32),
)


@functools.partial(
    pl.kernel,
    out_type=jax.ShapeDtypeStruct((NC * N, HB), jnp.float32),
    mesh=_mesh,
    compiler_params=pltpu.CompilerParams(use_tc_tiling_on_sc=False),
    scratch_types=[
        pltpu.VMEM_SHARED((N, HB), jnp.float32),
        pltpu.VMEM_SHARED((N, HB), jnp.float32),
        pltpu.VMEM((2, GC, CH), jnp.int32),
        pltpu.VMEM((2, GC, CH), jnp.int32),
        pltpu.VMEM((2, GC, CH), jnp.float32),
        pltpu.VMEM((DEPTH, CH, HB), jnp.float32),
        pltpu.VMEM((RC, HB), jnp.float32),
        pltpu.VMEM((RC, HB), jnp.float32),
        pltpu.VMEM((16,), jnp.float32),
        pltpu.SemaphoreType.DMA((DEPTH,)),
        pltpu.SemaphoreType.DMA((DEPTH,)),
        pltpu.SemaphoreType.DMA,
    ],
)
def _rlayer_sc(zuf, yf, rows_t, cols_t, vals_t, eta16,
               out, src_sh, acc_sh, gidx, sidx, valv, gbuf, sa, sb, etav,
               gsem, ssem, isem):
    c = lax.axis_index("c")
    s = lax.axis_index("s")
    base = s * SLAB
    hbase = c * N + base

    pltpu.sync_copy(eta16, etav)

    def _fill_zero(buf):
        zv = jnp.zeros((16,), jnp.float32)

        def body(r, _):
            buf[r, pl.ds(0, 16)] = zv
            buf[r, pl.ds(16, 16)] = zv
            return 0

        lax.fori_loop(0, RC, body, 0)

    _fill_zero(sb)
    for k in range(NRC):
        r0 = base + k * RC
        h0 = hbase + k * RC
        pltpu.sync_copy(zuf.at[pl.ds(h0, RC)], sa)
        pltpu.sync_copy(sa, src_sh.at[pl.ds(r0, RC)])
        pltpu.sync_copy(sb, acc_sh.at[pl.ds(r0, RC)])
    plsc.subcore_barrier()

    def spmm_pass(g_hbm, s_hbm, v_hbm, mul_eta):
        ev = etav[pl.ds(0, 16)]

        def wait_buf(sem_slice, dst):
            pltpu.make_async_copy(zuf.at[pl.ds(0, CH)], dst, sem_slice).wait()

        pltpu.async_copy(g_hbm.at[s, 0], gidx.at[0], isem)
        pltpu.async_copy(s_hbm.at[s, 0], sidx.at[0], isem)
        pltpu.async_copy(v_hbm.at[s, 0], valv.at[0], isem)

        def group_body(g, _):
            gset = lax.rem(g, 2)
            pltpu.make_async_copy(g_hbm.at[s, g], gidx.at[gset], isem).wait()
            pltpu.make_async_copy(s_hbm.at[s, g], sidx.at[gset], isem).wait()
            pltpu.make_async_copy(v_hbm.at[s, g], valv.at[gset], isem).wait()

            @pl.when(g < NG - 1)
            def _():
                nset = lax.rem(g + 1, 2)
                pltpu.async_copy(g_hbm.at[s, g + 1], gidx.at[nset], isem)
                pltpu.async_copy(s_hbm.at[s, g + 1], sidx.at[nset], isem)
                pltpu.async_copy(v_hbm.at[s, g + 1], valv.at[nset], isem)

            for p in range(DEPTH - 1):
                pltpu.async_copy(
                    src_sh.at[gidx.at[gset, p]], gbuf.at[p], gsem.at[p]
                )

            def slot(j, p):
                wait_buf(gsem.at[p], gbuf.at[p])

                @plsc.parallel_loop(0, CH // 16, 1, unroll=2)
                def scale_body(q):
                    base_cc = q * 16
                    vv = valv[gset, j, pl.ds(base_cc, 16)]
                    if mul_eta:
                        vv = vv * ev
                    for i in range(16):
                        v = vv[i]
                        cc = base_cc + i
                        gbuf[p, cc, pl.ds(0, 16)] = gbuf[p, cc, pl.ds(0, 16)] * v
                        gbuf[p, cc, pl.ds(16, 16)] = gbuf[p, cc, pl.ds(16, 16)] * v

                pltpu.async_copy(
                    gbuf.at[p], acc_sh.at[sidx.at[gset, j]], ssem.at[p],
                    add=True,
                )
                nxt = (p + DEPTH - 1) % DEPTH

                @pl.when(j >= 1)
                def _():
                    wait_buf(ssem.at[nxt], gbuf.at[nxt])

                @pl.when(j + DEPTH - 1 < GC)
                def _():
                    pltpu.async_copy(
                        src_sh.at[gidx.at[gset, j + DEPTH - 1]],
                        gbuf.at[nxt], gsem.at[nxt],
                    )

            def ring_body(kk, _):
                for p in range(DEPTH):
                    slot(kk * DEPTH + p, p)
                return 0

            lax.fori_loop(0, GC // DEPTH, ring_body, 0)
            wait_buf(ssem.at[(GC - 1) % DEPTH], gbuf.at[(GC - 1) % DEPTH])
            return 0

        lax.fori_loop(0, NG, group_body, 0)

    spmm_pass(cols_t, rows_t, vals_t, False)
    plsc.subcore_barrier()

    _fill_zero(sb)
    for k in range(NRC):
        r0 = base + k * RC
        h0 = hbase + k * RC
        pltpu.sync_copy(acc_sh.at[pl.ds(r0, RC)], sa)
        pltpu.sync_copy(yf.at[pl.ds(h0, RC)], gbuf.at[0])

        def sub_body(r, _):
            sa[r, pl.ds(0, 16)] = gbuf[0, r, pl.ds(0, 16)] - sa[r, pl.ds(0, 16)]
            sa[r, pl.ds(16, 16)] = gbuf[0, r, pl.ds(16, 16)] - sa[r, pl.ds(16, 16)]
            return 0

        lax.fori_loop(0, RC, sub_body, 0)
        pltpu.sync_copy(sa, src_sh.at[pl.ds(r0, RC)])
        pltpu.sync_copy(sb, acc_sh.at[pl.ds(r0, RC)])
    plsc.subcore_barrier()

    spmm_pass(rows_t, cols_t, vals_t, True)
    plsc.subcore_barrier()

    for k in range(NRC):
        r0 = base + k * RC
        h0 = hbase + k * RC
        pltpu.sync_copy(acc_sh.at[pl.ds(r0, RC)], sa)
        pltpu.sync_copy(sa, out.at[pl.ds(h0, RC)])


def kernel(z, u, y, A_vals, eta, A_rows, A_cols):
    zuf, yf = _tc_pre(z, u, y)
    rows_t = A_rows.reshape(NS, NG, GC, CH)
    cols_t = A_cols.reshape(NS, NG, GC, CH)
    vals_t = A_vals.reshape(NS, NG, GC, CH)
    eta16 = jnp.full((16,), eta, dtype=jnp.float32)
    es = _rlayer_sc(zuf, yf, rows_t, cols_t, vals_t, eta16)
    return _tc_post(es, z, u)


# speedup vs baseline: 1.1354x; 1.1354x over previous
import functools

import jax
import jax.numpy as jnp
from jax import lax
from jax.experimental import pallas as pl
from jax.experimental.pallas import tpu as pltpu
from jax.experimental.pallas import tpu_sc as plsc

N = 16384
NNZ = 262144
BATCH = 64
NC = 2
NS = 16
HB = BATCH // NC
NPT = NNZ // NS
CH = 128
NCH = NPT // CH
NG = 4
GC = NCH // NG
DEPTH = 4
SLAB = N // NS
RC = 128
NRC = SLAB // RC
TB = 512

_mesh = plsc.VectorSubcoreMesh(core_axis_name="c", subcore_axis_name="s")


def _tsp4(x):
    return jnp.concatenate(
        [x[:, q * 128:(q + 1) * 128].T for q in range(4)], axis=1
    )


def _pre_body(z_ref, u_ref, y_ref, zu_ref, yt_ref):
    zu = z_ref[...] * u_ref[...]
    zu_ref[...] = _tsp4(zu)
    yt_ref[...] = _tsp4(y_ref[...])


_raw_spec = pl.BlockSpec((HB, TB), lambda c, j: (c, j))
_tsp_spec = pl.BlockSpec((TB // 4, 4 * HB), lambda c, j: (c * (N // TB) + j, 0))

_tc_pre = pl.pallas_call(
    _pre_body,
    grid=(NC, N // TB),
    in_specs=[_raw_spec, _raw_spec, _raw_spec],
    out_specs=[_tsp_spec, _tsp_spec],
    out_shape=[
        jax.ShapeDtypeStruct((NC * N // 4, 4 * HB), jnp.float32),
        jax.ShapeDtypeStruct((NC * N // 4, 4 * HB), jnp.float32),
    ],
)


def _post_body(es_ref, z_ref, u_ref, o_ref):
    es = es_ref[...]
    es_t = jnp.concatenate(
        [es[:, q * HB:(q + 1) * HB].T for q in range(4)], axis=1
    )
    o_ref[...] = z_ref[...] - u_ref[...] * es_t


_tc_post = pl.pallas_call(
    _post_body,
    grid=(NC, N // TB),
    in_specs=[_tsp_spec, _raw_spec, _raw_spec],
    out_specs=_raw_spec,
    out_shape=jax.ShapeDtypeStruct((BATCH, N), jnp.float32),
)


@functools.partial(
    pl.kernel,
    out_type=jax.ShapeDtypeStruct((NC * N, HB), jnp.float32),
    mesh=_mesh,
    compiler_params=pltpu.CompilerParams(use_tc_tiling_on_sc=False),
    scratch_types=[
        pltpu.VMEM_SHARED((N, HB), jnp.float32),
        pltpu.VMEM_SHARED((N, HB), jnp.float32),
        pltpu.VMEM((2, GC, CH), jnp.int32),
        pltpu.VMEM((2, GC, CH), jnp.int32),
        pltpu.VMEM((2, GC, CH), jnp.float32),
        pltpu.VMEM((DEPTH, CH, HB), jnp.float32),
        pltpu.VMEM((RC, HB), jnp.float32),
        pltpu.VMEM((RC, HB), jnp.float32),
        pltpu.VMEM((16,), jnp.float32),
        pltpu.SemaphoreType.DMA((DEPTH,)),
        pltpu.SemaphoreType.DMA((DEPTH,)),
        pltpu.SemaphoreType.DMA,
    ],
)
def _rlayer_sc(zuf, yf, rows_t, cols_t, vals_t, eta16,
               out, src_sh, acc_sh, gidx, sidx, valv, gbuf, sa, sb, etav,
               gsem, ssem, isem):
    c = lax.axis_index("c")
    s = lax.axis_index("s")
    base = s * SLAB
    hbase = c * N + base

    pltpu.sync_copy(eta16, etav)

    def _fill_zero(buf):
        zv = jnp.zeros((16,), jnp.float32)

        def body(r, _):
            buf[r, pl.ds(0, 16)] = zv
            buf[r, pl.ds(16, 16)] = zv
            return 0

        lax.fori_loop(0, RC, body, 0)

    _fill_zero(sb)
    for k in range(NRC):
        r0 = base + k * RC
        h0 = hbase + k * RC
        pltpu.sync_copy(zuf.at[pl.ds(h0, RC)], sa)
        pltpu.sync_copy(sa, src_sh.at[pl.ds(r0, RC)])
        pltpu.sync_copy(sb, acc_sh.at[pl.ds(r0, RC)])
    plsc.subcore_barrier()

    def spmm_pass(g_hbm, s_hbm, v_hbm, mul_eta):
        ev = etav[pl.ds(0, 16)]

        def wait_buf(sem_slice, dst):
            pltpu.make_async_copy(zuf.at[pl.ds(0, CH)], dst, sem_slice).wait()

        pltpu.async_copy(g_hbm.at[s, 0], gidx.at[0], isem)
        pltpu.async_copy(s_hbm.at[s, 0], sidx.at[0], isem)
        pltpu.async_copy(v_hbm.at[s, 0], valv.at[0], isem)

        def group_body(g, _):
            gset = lax.rem(g, 2)
            pltpu.make_async_copy(g_hbm.at[s, g], gidx.at[gset], isem).wait()
            pltpu.make_async_copy(s_hbm.at[s, g], sidx.at[gset], isem).wait()
            pltpu.make_async_copy(v_hbm.at[s, g], valv.at[gset], isem).wait()

            @pl.when(g < NG - 1)
            def _():
                nset = lax.rem(g + 1, 2)
                pltpu.async_copy(g_hbm.at[s, g + 1], gidx.at[nset], isem)
                pltpu.async_copy(s_hbm.at[s, g + 1], sidx.at[nset], isem)
                pltpu.async_copy(v_hbm.at[s, g + 1], valv.at[nset], isem)

            for p in range(DEPTH - 1):
                pltpu.async_copy(
                    src_sh.at[gidx.at[gset, p]], gbuf.at[p], gsem.at[p]
                )

            def slot(j, p):
                wait_buf(gsem.at[p], gbuf.at[p])

                @plsc.parallel_loop(0, CH // 16, 1, unroll=2)
                def scale_body(q):
                    base_cc = q * 16
                    vv = valv[gset, j, pl.ds(base_cc, 16)]
                    if mul_eta:
                        vv = vv * ev
                    for i in range(16):
                        v = vv[i]
                        cc = base_cc + i
                        gbuf[p, cc, pl.ds(0, 16)] = gbuf[p, cc, pl.ds(0, 16)] * v
                        gbuf[p, cc, pl.ds(16, 16)] = gbuf[p, cc, pl.ds(16, 16)] * v

                pltpu.async_copy(
                    gbuf.at[p], acc_sh.at[sidx.at[gset, j]], ssem.at[p],
                    add=True,
                )
                nxt = (p + DEPTH - 1) % DEPTH

                @pl.when(j >= 1)
                def _():
                    wait_buf(ssem.at[nxt], gbuf.at[nxt])

                @pl.when(j + DEPTH - 1 < GC)
                def _():
                    pltpu.async_copy(
                        src_sh.at[gidx.at[gset, j + DEPTH - 1]],
                        gbuf.at[nxt], gsem.at[nxt],
                    )

            def ring_body(kk, _):
                for p in range(DEPTH):
                    slot(kk * DEPTH + p, p)
                return 0

            lax.fori_loop(0, GC // DEPTH, ring_body, 0)
            wait_buf(ssem.at[(GC - 1) % DEPTH], gbuf.at[(GC - 1) % DEPTH])
            return 0

        lax.fori_loop(0, NG, group_body, 0)

    spmm_pass(cols_t, rows_t, vals_t, False)
    plsc.subcore_barrier()

    _fill_zero(sb)
    for k in range(NRC):
        r0 = base + k * RC
        h0 = hbase + k * RC
        pltpu.sync_copy(acc_sh.at[pl.ds(r0, RC)], sa)
        pltpu.sync_copy(yf.at[pl.ds(h0, RC)], gbuf.at[0])

        def sub_body(r, _):
            sa[r, pl.ds(0, 16)] = gbuf[0, r, pl.ds(0, 16)] - sa[r, pl.ds(0, 16)]
            sa[r, pl.ds(16, 16)] = gbuf[0, r, pl.ds(16, 16)] - sa[r, pl.ds(16, 16)]
            return 0

        lax.fori_loop(0, RC, sub_body, 0)
        pltpu.sync_copy(sa, src_sh.at[pl.ds(r0, RC)])
        pltpu.sync_copy(sb, acc_sh.at[pl.ds(r0, RC)])
    plsc.subcore_barrier()

    spmm_pass(rows_t, cols_t, vals_t, True)
    plsc.subcore_barrier()

    for k in range(NRC):
        r0 = base + k * RC
        h0 = hbase + k * RC
        pltpu.sync_copy(acc_sh.at[pl.ds(r0, RC)], sa)
        pltpu.sync_copy(sa, out.at[pl.ds(h0, RC)])


def _perm(idx):
    return (idx & ~511) | ((idx & 127) << 2) | ((idx >> 7) & 3)


def kernel(z, u, y, A_vals, eta, A_rows, A_cols):
    zu4, y4 = _tc_pre(z, u, y)
    zuf = zu4.reshape(NC * N, HB)
    yf = y4.reshape(NC * N, HB)
    rows_t = _perm(A_rows).reshape(NS, NG, GC, CH)
    cols_t = _perm(A_cols).reshape(NS, NG, GC, CH)
    vals_t = A_vals.reshape(NS, NG, GC, CH)
    eta16 = jnp.full((16,), eta, dtype=jnp.float32)
    es = _rlayer_sc(zuf, yf, rows_t, cols_t, vals_t, eta16)
    return _tc_post(es.reshape(NC * N // 4, 4 * HB), z, u)
